# baseline (speedup 1.0000x reference)
import jax
import jax.numpy as jnp
from jax import lax
from jax.experimental import pallas as pl
from jax.experimental.pallas import tpu as pltpu
from jax.experimental.pallas import tpu_sc as plsc

BATCH = 16384
DIM = 64

_NC = 2
_NS = 16
_NW = _NC * _NS
_BPW = BATCH // _NW
_CHUNK = 128
_NCH = _BPW // _CHUNK


def _body(feat_a_hbm, feat_b_hbm, wa_hbm, wb_hbm, out_hbm,
          idx_v, a_v, b_v, sem):
    wid = lax.axis_index("s") * _NC + lax.axis_index("c")
    base = wid * _BPW

    for j in range(_NCH):
        pltpu.sync_copy(feat_a_hbm.at[pl.ds(base + j * _CHUNK, _CHUNK)],
                        idx_v.at[0, j])
        pltpu.sync_copy(feat_b_hbm.at[pl.ds(base + j * _CHUNK, _CHUNK)],
                        idx_v.at[1, j])

    copies = []
    for j in range(_NCH):
        copies.append(pltpu.async_copy(
            wa_hbm.at[idx_v.at[0, j]],
            a_v.at[pl.ds(j * _CHUNK, _CHUNK)], sem))
        copies.append(pltpu.async_copy(
            wb_hbm.at[idx_v.at[1, j]],
            b_v.at[pl.ds(j * _CHUNK, _CHUNK)], sem))
    for c in copies:
        c.wait()

    pltpu.sync_copy(a_v, out_hbm.at[pl.ds(base, _BPW), 0])
    pltpu.sync_copy(b_v, out_hbm.at[pl.ds(base, _BPW), 1])


@jax.jit
def kernel(feat_a, feat_b, W_a, W_b):
    mesh = plsc.VectorSubcoreMesh(core_axis_name="c", subcore_axis_name="s")
    out = pl.kernel(
        _body,
        mesh=mesh,
        out_type=jax.ShapeDtypeStruct((BATCH, 2, DIM), jnp.float32),
        scratch_types=[
            pltpu.VMEM((2, _NCH, _CHUNK), jnp.int32),
            pltpu.VMEM((_BPW, DIM), jnp.float32),
            pltpu.VMEM((_BPW, DIM), jnp.float32),
            pltpu.SemaphoreType.DMA,
        ],
        compiler_params=pltpu.CompilerParams(use_tc_tiling_on_sc=False),
    )(feat_a, feat_b, W_a, W_b)
    return out.reshape(BATCH, 2 * DIM)


# speedup vs baseline: 1.6574x; 1.6574x over previous
import jax
import jax.numpy as jnp
from jax import lax
from jax.experimental import pallas as pl
from jax.experimental.pallas import tpu as pltpu
from jax.experimental.pallas import tpu_sc as plsc

BATCH = 16384
VOCAB = 1000000
DIM = 64

_NC = 2
_NS = 16
_NW = _NC * _NS
_BPW = BATCH // _NW
_L = 16
_K = 32
_NCHUNK = _BPW // _K


def _body(feat_a_hbm, feat_b_hbm, wa_hbm, wb_hbm, out_hbm,
          idxa_v, idxb_v, rows_v, sem):
    wid = lax.axis_index("s") * _NC + lax.axis_index("c")
    base = wid * _BPW

    pltpu.sync_copy(feat_a_hbm.at[pl.ds(base, _BPW)], idxa_v)
    pltpu.sync_copy(feat_b_hbm.at[pl.ds(base, _BPW)], idxb_v)

    lane = lax.iota(jnp.int32, _L)

    def chunk(c, _):
        k0 = c * _K
        copies = []
        for g in range(_K // _L):
            va = idxa_v[pl.ds(k0 + g * _L, _L)]
            vb = idxb_v[pl.ds(k0 + g * _L, _L)]
            for j in range(_L):
                slot = g * _L + j
                ra = jnp.sum(jnp.where(lane == j, va, 0))
                copies.append(pltpu.async_copy(
                    wa_hbm.at[ra, :],
                    rows_v.at[k0 + slot, pl.ds(0, DIM)], sem))
                rb = jnp.sum(jnp.where(lane == j, vb, 0))
                copies.append(pltpu.async_copy(
                    wb_hbm.at[rb, :],
                    rows_v.at[k0 + slot, pl.ds(DIM, DIM)], sem))
        for cp in copies:
            cp.wait()
        return ()

    lax.fori_loop(0, _NCHUNK, chunk, ())

    pltpu.sync_copy(rows_v, out_hbm.at[pl.ds(base, _BPW)])


@jax.jit
def kernel(feat_a, feat_b, W_a, W_b):
    mesh = plsc.VectorSubcoreMesh(core_axis_name="c", subcore_axis_name="s")
    out = pl.kernel(
        _body,
        mesh=mesh,
        out_type=jax.ShapeDtypeStruct((BATCH, 2 * DIM), jnp.float32),
        scratch_types=[
            pltpu.VMEM((_BPW,), jnp.int32),
            pltpu.VMEM((_BPW,), jnp.int32),
            pltpu.VMEM((_BPW, 2 * DIM), jnp.float32),
            pltpu.SemaphoreType.DMA,
        ],
        compiler_params=pltpu.CompilerParams(needs_layout_passes=False),
    )(feat_a, feat_b, W_a, W_b)
    return out


# speedup vs baseline: 1.6596x; 1.0013x over previous
import jax
import jax.numpy as jnp
from jax import lax
from jax.experimental import pallas as pl
from jax.experimental.pallas import tpu as pltpu
from jax.experimental.pallas import tpu_sc as plsc

BATCH = 16384
VOCAB = 1000000
DIM = 64

_NC = 2
_NS = 16
_NW = _NC * _NS
_BPW = BATCH // _NW
_L = 16
_K = 32
_NCHUNK = _BPW // _K


def _body(feat_a_hbm, feat_b_hbm, wa_hbm, wb_hbm, out_hbm,
          idxa_v, idxb_v, rows_v, sem):
    wid = lax.axis_index("s") * _NC + lax.axis_index("c")
    base = wid * _BPW

    pltpu.sync_copy(feat_a_hbm.at[pl.ds(base, _BPW)], idxa_v)
    pltpu.sync_copy(feat_b_hbm.at[pl.ds(base, _BPW)], idxb_v)

    def chunk(c, _):
        k0 = c * _K
        copies = []
        for g in range(_K // _L):
            va = idxa_v[pl.ds(k0 + g * _L, _L)]
            vb = idxb_v[pl.ds(k0 + g * _L, _L)]
            for j in range(_L):
                slot = g * _L + j
                copies.append(pltpu.async_copy(
                    wa_hbm.at[va[j], :],
                    rows_v.at[k0 + slot, pl.ds(0, DIM)], sem))
                copies.append(pltpu.async_copy(
                    wb_hbm.at[vb[j], :],
                    rows_v.at[k0 + slot, pl.ds(DIM, DIM)], sem))
        for cp in copies:
            cp.wait()
        return ()

    lax.fori_loop(0, _NCHUNK, chunk, ())

    pltpu.sync_copy(rows_v, out_hbm.at[pl.ds(base, _BPW)])


@jax.jit
def kernel(feat_a, feat_b, W_a, W_b):
    mesh = plsc.VectorSubcoreMesh(core_axis_name="c", subcore_axis_name="s")
    out = pl.kernel(
        _body,
        mesh=mesh,
        out_type=jax.ShapeDtypeStruct((BATCH, 2 * DIM), jnp.float32),
        scratch_types=[
            pltpu.VMEM((_BPW,), jnp.int32),
            pltpu.VMEM((_BPW,), jnp.int32),
            pltpu.VMEM((_BPW, 2 * DIM), jnp.float32),
            pltpu.SemaphoreType.DMA,
        ],
    )(feat_a, feat_b, W_a, W_b)
    return out
